# baseline (speedup 1.0000x reference)
import functools

import jax
import jax.numpy as jnp
from jax import lax
from jax.experimental import pallas as pl
from jax.experimental.pallas import tpu as pltpu
from jax.experimental.pallas import tpu_sc as plsc

_N = 10000
_NP = 10240
_E = 320000
_D = 128
_G = 64
_NC = 2
_NS = 16
_NW = _NC * _NS
_C = 128
_ZR = 128
_CHUNKS = 80
_PER_W = _CHUNKS * _C
_EP = _PER_W * _NW
_RPS = _NP // _NS


def _sc_aggregate(table, src_p, dst_p, zeros_tile):
    mesh = plsc.VectorSubcoreMesh(core_axis_name="c", subcore_axis_name="s")

    def body(table_h, src_h, dst_h, zeros_h, agg_o, agg_sh, src_v, dst_v,
             rows_v):
        c = lax.axis_index("c")
        s = lax.axis_index("s")

        @pl.loop(0, _RPS // _ZR)
        def _(j):
            pltpu.sync_copy(zeros_h, agg_sh.at[pl.ds(s * _RPS + j * _ZR, _ZR)])

        plsc.subcore_barrier()

        base = (c * _NS + s) * _PER_W

        @pl.loop(0, _CHUNKS)
        def _(i):
            off = base + i * _C
            pltpu.sync_copy(src_h.at[pl.ds(off, _C)], src_v.at[0])
            pltpu.sync_copy(dst_h.at[pl.ds(off, _C)], dst_v.at[0])
            pltpu.sync_copy(table_h.at[src_v.at[0]], rows_v)
            pltpu.sync_copy(rows_v, agg_sh.at[dst_v.at[0]], add=True)

        plsc.subcore_barrier()
        pltpu.sync_copy(agg_sh.at[pl.ds(s * _RPS, _RPS)],
                        agg_o.at[c, pl.ds(s * _RPS, _RPS)])

    run = pl.kernel(
        body,
        out_type=[jax.ShapeDtypeStruct((_NC, _NP, _D), jnp.float32)],
        mesh=mesh,
        scratch_types=[
            pltpu.VMEM_SHARED((_NP, _D), jnp.float32),
            pltpu.VMEM((1, _C), jnp.int32),
            pltpu.VMEM((1, _C), jnp.int32),
            pltpu.VMEM((_C, _D), jnp.float32),
        ])
    (agg,) = run(table, src_p, dst_p, zeros_tile)
    return agg


def _sc_degree(dst2d, zeros_tile, ones_tile):
    mesh = plsc.VectorSubcoreMesh(core_axis_name="c", subcore_axis_name="s")

    def body(dst_h, zeros_h, ones_h, cnt_o, cnt_sh, dst_v, ones_v, ssem):
        c = lax.axis_index("c")
        s = lax.axis_index("s")

        @pl.loop(0, _RPS // _C)
        def _(j):
            pltpu.sync_copy(zeros_h, cnt_sh.at[pl.ds(s * _RPS + j * _C, _C)])

        pltpu.sync_copy(ones_h, ones_v)
        wbase = (c * _NS + s) * _CHUNKS
        pltpu.sync_copy(dst_h.at[pl.ds(wbase, _CHUNKS)], dst_v)
        plsc.subcore_barrier()

        @pl.loop(0, _CHUNKS)
        def _(i):
            pltpu.async_copy(ones_v, cnt_sh.at[dst_v.at[i]], ssem, add=True)

        @pl.loop(0, _CHUNKS)
        def _(i):
            pltpu.make_async_copy(
                ones_v, cnt_sh.at[dst_v.at[0]], ssem).wait()

        plsc.subcore_barrier()
        pltpu.sync_copy(cnt_sh.at[pl.ds(s * _RPS, _RPS)],
                        cnt_o.at[c, pl.ds(s * _RPS, _RPS)])

    run = pl.kernel(
        body,
        out_type=[jax.ShapeDtypeStruct((_NC, _NP, _D), jnp.float32)],
        mesh=mesh,
        scratch_types=[
            pltpu.VMEM_SHARED((_NP, _D), jnp.float32),
            pltpu.VMEM((_CHUNKS, _C), jnp.int32),
            pltpu.VMEM((_C, _D), jnp.float32),
            pltpu.SemaphoreType.DMA,
        ])
    (cnt,) = run(dst2d, zeros_tile, ones_tile)
    return cnt


def _tc_layer_body(h_ref, a0_ref, a1_ref, c0_ref, c1_ref, wl_ref, wr_ref,
                   b_ref, o_ref):
    cnt = c0_ref[:, 0:1] + c1_ref[:, 0:1]
    scale = 1.0 / jnp.maximum(cnt, 1.0)
    mean = (a0_ref[...] + a1_ref[...]) * scale
    acc = jnp.dot(mean, wl_ref[...], preferred_element_type=jnp.float32)
    acc += jnp.dot(h_ref[...], wr_ref[...], preferred_element_type=jnp.float32)
    o_ref[...] = jnp.maximum(acc + b_ref[...], 0.0)


def _tc_layer(h, a0, a1, c0, c1, Wl, Wr, b):
    blk = 1024
    grid = _NP // blk
    return pl.pallas_call(
        _tc_layer_body,
        grid=(grid,),
        in_specs=[
            pl.BlockSpec((blk, _D), lambda i: (i, 0)),
            pl.BlockSpec((blk, _D), lambda i: (i, 0)),
            pl.BlockSpec((blk, _D), lambda i: (i, 0)),
            pl.BlockSpec((blk, _D), lambda i: (i, 0)),
            pl.BlockSpec((blk, _D), lambda i: (i, 0)),
            pl.BlockSpec((_D, _D), lambda i: (0, 0)),
            pl.BlockSpec((_D, _D), lambda i: (0, 0)),
            pl.BlockSpec((1, _D), lambda i: (0, 0)),
        ],
        out_specs=pl.BlockSpec((blk, _D), lambda i: (i, 0)),
        out_shape=jax.ShapeDtypeStruct((_NP, _D), jnp.float32),
    )(h, a0, a1, c0, c1, Wl, Wr, b)


def _tc_pool_head_body(h_ref, brow_ref, bcol_ref, w1_ref, b1_ref, w2_ref,
                       b2_ref, o_ref, add_acc, max_acc, cnt_acc):
    i = pl.program_id(0)
    nsteps = pl.num_programs(0)

    @pl.when(i == 0)
    def _():
        add_acc[...] = jnp.zeros_like(add_acc)
        cnt_acc[...] = jnp.zeros_like(cnt_acc)
        max_acc[...] = jnp.full_like(max_acc, -3.0e38)

    h = h_ref[...]
    brow = brow_ref[0]
    onehot = jnp.where(
        brow == lax.broadcasted_iota(jnp.int32, (_G, brow.shape[1]), 0),
        1.0, 0.0)
    add_acc[...] += jnp.dot(onehot, h, preferred_element_type=jnp.float32)
    cnt_acc[...] += jnp.sum(onehot, axis=1, keepdims=True)

    bmin = brow[0, 0]
    bmax = brow[0, brow.shape[1] - 1]
    bcol = bcol_ref[...]
    for g in range(_G):
        @pl.when((jnp.int32(g) >= bmin) & (jnp.int32(g) <= bmax))
        def _(g=g):
            masked = jnp.where(bcol == g, h, -3.0e38)
            mx = jnp.max(masked, axis=0, keepdims=True)
            max_acc[g:g + 1, :] = jnp.maximum(max_acc[g:g + 1, :], mx)

    @pl.when(i == nsteps - 1)
    def _():
        cnt = cnt_acc[:, 0:1]
        meanp = add_acc[...] / jnp.maximum(cnt, 1.0)
        mx = jnp.where(cnt > 0, max_acc[...], 0.0)
        w1 = w1_ref[...]
        z = jnp.dot(add_acc[...], w1[0:_D, :],
                    preferred_element_type=jnp.float32)
        z += jnp.dot(meanp, w1[_D:2 * _D, :],
                     preferred_element_type=jnp.float32)
        z += jnp.dot(mx, w1[2 * _D:3 * _D, :],
                     preferred_element_type=jnp.float32)
        z = jnp.maximum(z + b1_ref[...], 0.0)
        o = jnp.dot(z, w2_ref[...], preferred_element_type=jnp.float32)
        o += b2_ref[...]
        m = jnp.max(o, axis=1, keepdims=True)
        sh = o - m
        lse = jnp.log(jnp.sum(jnp.exp(sh), axis=1, keepdims=True))
        o_ref[...] = sh - lse


def _tc_pool_head(h, brow, bcol, W1, b1, W2, b2):
    blk = 1024
    grid = _NP // blk
    return pl.pallas_call(
        _tc_pool_head_body,
        grid=(grid,),
        in_specs=[
            pl.BlockSpec((blk, _D), lambda i: (i, 0)),
            pl.BlockSpec((1, 1, blk), lambda i: (i, 0, 0)),
            pl.BlockSpec((blk, 1), lambda i: (i, 0)),
            pl.BlockSpec((3 * _D, _D), lambda i: (0, 0)),
            pl.BlockSpec((1, _D), lambda i: (0, 0)),
            pl.BlockSpec((_D, 2), lambda i: (0, 0)),
            pl.BlockSpec((1, 2), lambda i: (0, 0)),
        ],
        out_specs=pl.BlockSpec((_G, 2), lambda i: (0, 0)),
        out_shape=jax.ShapeDtypeStruct((_G, 2), jnp.float32),
        scratch_shapes=[
            pltpu.VMEM((_G, _D), jnp.float32),
            pltpu.VMEM((_G, _D), jnp.float32),
            pltpu.VMEM((_G, 1), jnp.float32),
        ],
    )(h, brow, bcol, W1, b1, W2, b2)


def kernel(x, edge_index, batch, Wl1, Wr1, b1, Wl2, Wr2, b2, Wl3, Wr3, b3,
           W_lin1, b_lin1, W_lin2, b_lin2):
    f32 = jnp.float32
    x = x.astype(f32)

    x_pad = jnp.zeros((_NP, _D), f32).at[:_N].set(x)
    src = edge_index[0].astype(jnp.int32)
    dst = edge_index[1].astype(jnp.int32)
    npad = _EP - _E
    src_p = jnp.concatenate([src, jnp.zeros((npad,), jnp.int32)])
    pad_dst = _N + (jnp.arange(npad, dtype=jnp.int32) % (_NP - _N))
    dst_p = jnp.concatenate([dst, pad_dst])
    src2d = src_p.reshape(_EP // _C, _C)
    dst2d = dst_p.reshape(_EP // _C, _C)
    batch_p = jnp.concatenate(
        [batch.astype(jnp.int32), jnp.full((_NP - _N,), _G, jnp.int32)])
    brow = batch_p.reshape(_NP // 1024, 1, 1024)
    bcol = batch_p.reshape(_NP, 1)
    zerosZ = jnp.zeros((_ZR, _D), f32)
    zerosC = jnp.zeros((_C, _D), f32)
    onesC = jnp.ones((_C, _D), f32)
    b1r = b1.reshape(1, _D).astype(f32)
    b2r = b2.reshape(1, _D).astype(f32)
    b3r = b3.reshape(1, _D).astype(f32)
    bl1 = b_lin1.reshape(1, _D).astype(f32)
    bl2 = b_lin2.reshape(1, 2).astype(f32)

    cnt = _sc_degree(dst2d, zerosC, onesC)
    c0, c1 = cnt[0], cnt[1]

    agg = _sc_aggregate(x_pad, src_p, dst_p, zerosZ)
    h = _tc_layer(x_pad, agg[0], agg[1], c0, c1, Wl1, Wr1, b1r)

    agg = _sc_aggregate(h, src_p, dst_p, zerosZ)
    h = _tc_layer(h, agg[0], agg[1], c0, c1, Wl2, Wr2, b2r)

    agg = _sc_aggregate(h, src_p, dst_p, zerosZ)
    h = _tc_layer(h, agg[0], agg[1], c0, c1, Wl3, Wr3, b3r)

    return _tc_pool_head(h, brow, bcol, W_lin1, bl1, W_lin2, bl2)


# speedup vs baseline: 1.4674x; 1.4674x over previous
import functools

import jax
import jax.numpy as jnp
from jax import lax
from jax.experimental import pallas as pl
from jax.experimental.pallas import tpu as pltpu
from jax.experimental.pallas import tpu_sc as plsc

_N = 10000
_NP = 10240
_E = 320000
_D = 128
_G = 64
_NC = 2
_NS = 16
_NW = _NC * _NS
_C = 128
_ZR = 128
_EP = ((_E + _NW * _C - 1) // (_NW * _C)) * (_NW * _C)
_PER_W = _EP // _NW
_CHUNKS = _PER_W // _C
_RPS = _NP // _NS


def _sc_aggregate(table, src_p, dst_p, zeros_tile):
    mesh = plsc.VectorSubcoreMesh(core_axis_name="c", subcore_axis_name="s")

    def body(table_h, src_h, dst_h, zeros_h, agg_o, agg_sh, src_v, dst_v,
             rows_v):
        c = lax.axis_index("c")
        s = lax.axis_index("s")

        @pl.loop(0, _RPS // _ZR)
        def _(j):
            pltpu.sync_copy(zeros_h, agg_sh.at[pl.ds(s * _RPS + j * _ZR, _ZR)])

        plsc.subcore_barrier()

        base = (c * _NS + s) * _PER_W

        @pl.loop(0, _CHUNKS)
        def _(i):
            off = base + i * _C
            pltpu.sync_copy(src_h.at[pl.ds(off, _C)], src_v.at[0])
            pltpu.sync_copy(dst_h.at[pl.ds(off, _C)], dst_v.at[0])
            pltpu.sync_copy(table_h.at[src_v.at[0]], rows_v)
            pltpu.sync_copy(rows_v, agg_sh.at[dst_v.at[0]], add=True)

        plsc.subcore_barrier()
        pltpu.sync_copy(agg_sh.at[pl.ds(s * _RPS, _RPS)],
                        agg_o.at[c, pl.ds(s * _RPS, _RPS)])

    run = pl.kernel(
        body,
        out_type=[jax.ShapeDtypeStruct((_NC, _NP, _D), jnp.float32)],
        mesh=mesh,
        scratch_types=[
            pltpu.VMEM_SHARED((_NP, _D), jnp.float32),
            pltpu.VMEM((1, _C), jnp.int32),
            pltpu.VMEM((1, _C), jnp.int32),
            pltpu.VMEM((_C, _D), jnp.float32),
        ])
    (agg,) = run(table, src_p, dst_p, zeros_tile)
    return agg


def _sc_degree(dst_p, zeros_tile, ones_tile):
    mesh = plsc.VectorSubcoreMesh(core_axis_name="c", subcore_axis_name="s")

    def body(dst_h, zeros_h, ones_h, cnt_o, cnt_sh, dst_v, ones_v):
        c = lax.axis_index("c")
        s = lax.axis_index("s")

        @pl.loop(0, _RPS // _C)
        def _(j):
            pltpu.sync_copy(zeros_h, cnt_sh.at[pl.ds(s * _RPS + j * _C, _C)])

        pltpu.sync_copy(ones_h, ones_v)
        plsc.subcore_barrier()

        base = (c * _NS + s) * _PER_W

        @pl.loop(0, _CHUNKS)
        def _(i):
            pltpu.sync_copy(dst_h.at[pl.ds(base + i * _C, _C)], dst_v.at[0])
            pltpu.sync_copy(ones_v, cnt_sh.at[dst_v.at[0]], add=True)

        plsc.subcore_barrier()
        pltpu.sync_copy(cnt_sh.at[pl.ds(s * _RPS, _RPS)],
                        cnt_o.at[c, pl.ds(s * _RPS, _RPS)])

    run = pl.kernel(
        body,
        out_type=[jax.ShapeDtypeStruct((_NC, _NP, _D), jnp.float32)],
        mesh=mesh,
        scratch_types=[
            pltpu.VMEM_SHARED((_NP, _D), jnp.float32),
            pltpu.VMEM((1, _C), jnp.int32),
            pltpu.VMEM((_C, _D), jnp.float32),
        ])
    (cnt,) = run(dst_p, zeros_tile, ones_tile)
    return cnt


def _tc_layer_body(h_ref, a0_ref, a1_ref, c0_ref, c1_ref, wl_ref, wr_ref,
                   b_ref, o_ref):
    cnt = c0_ref[:, 0:1] + c1_ref[:, 0:1]
    scale = 1.0 / jnp.maximum(cnt, 1.0)
    mean = (a0_ref[...] + a1_ref[...]) * scale
    acc = jnp.dot(mean, wl_ref[...], preferred_element_type=jnp.float32)
    acc += jnp.dot(h_ref[...], wr_ref[...], preferred_element_type=jnp.float32)
    o_ref[...] = jnp.maximum(acc + b_ref[...], 0.0)


def _tc_layer(h, a0, a1, c0, c1, Wl, Wr, b):
    blk = 1024
    grid = _NP // blk
    return pl.pallas_call(
        _tc_layer_body,
        grid=(grid,),
        in_specs=[
            pl.BlockSpec((blk, _D), lambda i: (i, 0)),
            pl.BlockSpec((blk, _D), lambda i: (i, 0)),
            pl.BlockSpec((blk, _D), lambda i: (i, 0)),
            pl.BlockSpec((blk, _D), lambda i: (i, 0)),
            pl.BlockSpec((blk, _D), lambda i: (i, 0)),
            pl.BlockSpec((_D, _D), lambda i: (0, 0)),
            pl.BlockSpec((_D, _D), lambda i: (0, 0)),
            pl.BlockSpec((1, _D), lambda i: (0, 0)),
        ],
        out_specs=pl.BlockSpec((blk, _D), lambda i: (i, 0)),
        out_shape=jax.ShapeDtypeStruct((_NP, _D), jnp.float32),
    )(h, a0, a1, c0, c1, Wl, Wr, b)


def _tc_pool_head_body(h_ref, brow_ref, bcol_ref, w1_ref, b1_ref, w2_ref,
                       b2_ref, o_ref, add_acc, max_acc, cnt_acc):
    i = pl.program_id(0)
    nsteps = pl.num_programs(0)

    @pl.when(i == 0)
    def _():
        add_acc[...] = jnp.zeros_like(add_acc)
        cnt_acc[...] = jnp.zeros_like(cnt_acc)
        max_acc[...] = jnp.full_like(max_acc, -3.0e38)

    h = h_ref[...]
    brow = brow_ref[0]
    onehot = jnp.where(
        brow == lax.broadcasted_iota(jnp.int32, (_G, brow.shape[1]), 0),
        1.0, 0.0)
    add_acc[...] += jnp.dot(onehot, h, preferred_element_type=jnp.float32)
    cnt_acc[...] += jnp.sum(onehot, axis=1, keepdims=True)

    bmin = brow[0, 0]
    bmax = brow[0, brow.shape[1] - 1]
    bcol = bcol_ref[...]
    for g in range(_G):
        @pl.when((jnp.int32(g) >= bmin) & (jnp.int32(g) <= bmax))
        def _(g=g):
            masked = jnp.where(bcol == g, h, -3.0e38)
            mx = jnp.max(masked, axis=0, keepdims=True)
            max_acc[g:g + 1, :] = jnp.maximum(max_acc[g:g + 1, :], mx)

    @pl.when(i == nsteps - 1)
    def _():
        cnt = cnt_acc[:, 0:1]
        meanp = add_acc[...] / jnp.maximum(cnt, 1.0)
        mx = jnp.where(cnt > 0, max_acc[...], 0.0)
        w1 = w1_ref[...]
        z = jnp.dot(add_acc[...], w1[0:_D, :],
                    preferred_element_type=jnp.float32)
        z += jnp.dot(meanp, w1[_D:2 * _D, :],
                     preferred_element_type=jnp.float32)
        z += jnp.dot(mx, w1[2 * _D:3 * _D, :],
                     preferred_element_type=jnp.float32)
        z = jnp.maximum(z + b1_ref[...], 0.0)
        o = jnp.dot(z, w2_ref[...], preferred_element_type=jnp.float32)
        o += b2_ref[...]
        m = jnp.max(o, axis=1, keepdims=True)
        sh = o - m
        lse = jnp.log(jnp.sum(jnp.exp(sh), axis=1, keepdims=True))
        o_ref[...] = sh - lse


def _tc_pool_head(h, brow, bcol, W1, b1, W2, b2):
    blk = 1024
    grid = _NP // blk
    return pl.pallas_call(
        _tc_pool_head_body,
        grid=(grid,),
        in_specs=[
            pl.BlockSpec((blk, _D), lambda i: (i, 0)),
            pl.BlockSpec((1, 1, blk), lambda i: (i, 0, 0)),
            pl.BlockSpec((blk, 1), lambda i: (i, 0)),
            pl.BlockSpec((3 * _D, _D), lambda i: (0, 0)),
            pl.BlockSpec((1, _D), lambda i: (0, 0)),
            pl.BlockSpec((_D, 2), lambda i: (0, 0)),
            pl.BlockSpec((1, 2), lambda i: (0, 0)),
        ],
        out_specs=pl.BlockSpec((_G, 2), lambda i: (0, 0)),
        out_shape=jax.ShapeDtypeStruct((_G, 2), jnp.float32),
        scratch_shapes=[
            pltpu.VMEM((_G, _D), jnp.float32),
            pltpu.VMEM((_G, _D), jnp.float32),
            pltpu.VMEM((_G, 1), jnp.float32),
        ],
    )(h, brow, bcol, W1, b1, W2, b2)


def kernel(x, edge_index, batch, Wl1, Wr1, b1, Wl2, Wr2, b2, Wl3, Wr3, b3,
           W_lin1, b_lin1, W_lin2, b_lin2):
    f32 = jnp.float32
    x = x.astype(f32)

    x_pad = jnp.zeros((_NP, _D), f32).at[:_N].set(x)
    src = edge_index[0].astype(jnp.int32)
    dst = edge_index[1].astype(jnp.int32)
    npad = _EP - _E
    src_p = jnp.concatenate([src, jnp.zeros((npad,), jnp.int32)])
    dst_p = jnp.concatenate([dst, jnp.full((npad,), _N, jnp.int32)])
    batch_p = jnp.concatenate(
        [batch.astype(jnp.int32), jnp.full((_NP - _N,), _G, jnp.int32)])
    brow = batch_p.reshape(_NP // 1024, 1, 1024)
    bcol = batch_p.reshape(_NP, 1)
    zerosZ = jnp.zeros((_ZR, _D), f32)
    zerosC = jnp.zeros((_C, _D), f32)
    onesC = jnp.ones((_C, _D), f32)
    b1r = b1.reshape(1, _D).astype(f32)
    b2r = b2.reshape(1, _D).astype(f32)
    b3r = b3.reshape(1, _D).astype(f32)
    bl1 = b_lin1.reshape(1, _D).astype(f32)
    bl2 = b_lin2.reshape(1, 2).astype(f32)

    cnt = _sc_degree(dst_p, zerosC, onesC)
    c0, c1 = cnt[0], cnt[1]

    agg = _sc_aggregate(x_pad, src_p, dst_p, zerosZ)
    h = _tc_layer(x_pad, agg[0], agg[1], c0, c1, Wl1, Wr1, b1r)

    agg = _sc_aggregate(h, src_p, dst_p, zerosZ)
    h = _tc_layer(h, agg[0], agg[1], c0, c1, Wl2, Wr2, b2r)

    agg = _sc_aggregate(h, src_p, dst_p, zerosZ)
    h = _tc_layer(h, agg[0], agg[1], c0, c1, Wl3, Wr3, b3r)

    return _tc_pool_head(h, brow, bcol, W_lin1, bl1, W_lin2, bl2)


# speedup vs baseline: 1.6271x; 1.1088x over previous
import functools

import jax
import jax.numpy as jnp
from jax import lax
from jax.experimental import pallas as pl
from jax.experimental.pallas import tpu as pltpu
from jax.experimental.pallas import tpu_sc as plsc

_N = 10000
_NP = 10240
_E = 320000
_D = 128
_G = 64
_NC = 2
_NS = 16
_NW = _NC * _NS
_C = 128
_ZR = 128
_EP = ((_E + _NW * _C - 1) // (_NW * _C)) * (_NW * _C)
_PER_W = _EP // _NW
_CHUNKS = _PER_W // _C
_RPS = _NP // _NS


def _sc_aggregate(table, ei2d, zeros_tile):
    mesh = plsc.VectorSubcoreMesh(core_axis_name="c", subcore_axis_name="s")

    def body(table_h, ei_h, zeros_h, agg_o, agg_sh, idx_v, rows_v):
        c = lax.axis_index("c")
        s = lax.axis_index("s")

        @pl.loop(0, _RPS // _ZR)
        def _(j):
            pltpu.sync_copy(zeros_h, agg_sh.at[pl.ds(s * _RPS + j * _ZR, _ZR)])

        plsc.subcore_barrier()

        base = (c * _NS + s) * 2 * _CHUNKS

        @pl.loop(0, _CHUNKS)
        def _(i):
            pltpu.sync_copy(ei_h.at[pl.ds(base + 2 * i, 2)], idx_v)
            pltpu.sync_copy(table_h.at[idx_v.at[0]], rows_v)
            pltpu.sync_copy(rows_v, agg_sh.at[idx_v.at[1]], add=True)

        plsc.subcore_barrier()
        pltpu.sync_copy(agg_sh.at[pl.ds(s * _RPS, _RPS)],
                        agg_o.at[c, pl.ds(s * _RPS, _RPS)])

    run = pl.kernel(
        body,
        out_type=[jax.ShapeDtypeStruct((_NC, _NP, _D), jnp.float32)],
        mesh=mesh,
        scratch_types=[
            pltpu.VMEM_SHARED((_NP, _D), jnp.float32),
            pltpu.VMEM((2, _C), jnp.int32),
            pltpu.VMEM((_C, _D), jnp.float32),
        ])
    (agg,) = run(table, ei2d, zeros_tile)
    return agg


def _sc_degree(dst_p, zeros_tile, ones_tile):
    mesh = plsc.VectorSubcoreMesh(core_axis_name="c", subcore_axis_name="s")

    def body(dst_h, zeros_h, ones_h, cnt_o, cnt_sh, dst_v, ones_v):
        c = lax.axis_index("c")
        s = lax.axis_index("s")

        @pl.loop(0, _RPS // _C)
        def _(j):
            pltpu.sync_copy(zeros_h, cnt_sh.at[pl.ds(s * _RPS + j * _C, _C)])

        pltpu.sync_copy(ones_h, ones_v)
        plsc.subcore_barrier()

        base = (c * _NS + s) * _PER_W

        @pl.loop(0, _CHUNKS)
        def _(i):
            pltpu.sync_copy(dst_h.at[pl.ds(base + i * _C, _C)], dst_v.at[0])
            pltpu.sync_copy(ones_v, cnt_sh.at[dst_v.at[0]], add=True)

        plsc.subcore_barrier()
        pltpu.sync_copy(cnt_sh.at[pl.ds(s * _RPS, _RPS)],
                        cnt_o.at[c, pl.ds(s * _RPS, _RPS)])

    run = pl.kernel(
        body,
        out_type=[jax.ShapeDtypeStruct((_NC, _NP, _D), jnp.float32)],
        mesh=mesh,
        scratch_types=[
            pltpu.VMEM_SHARED((_NP, _D), jnp.float32),
            pltpu.VMEM((1, _C), jnp.int32),
            pltpu.VMEM((_C, _D), jnp.float32),
        ])
    (cnt,) = run(dst_p, zeros_tile, ones_tile)
    return cnt


def _tc_layer_body(h_ref, a0_ref, a1_ref, c0_ref, c1_ref, wl_ref, wr_ref,
                   b_ref, o_ref):
    cnt = c0_ref[:, 0:1] + c1_ref[:, 0:1]
    scale = 1.0 / jnp.maximum(cnt, 1.0)
    mean = (a0_ref[...] + a1_ref[...]) * scale
    acc = jnp.dot(mean, wl_ref[...], preferred_element_type=jnp.float32)
    acc += jnp.dot(h_ref[...], wr_ref[...], preferred_element_type=jnp.float32)
    o_ref[...] = jnp.maximum(acc + b_ref[...], 0.0)


def _tc_layer(h, a0, a1, c0, c1, Wl, Wr, b):
    blk = 1024
    grid = _NP // blk
    return pl.pallas_call(
        _tc_layer_body,
        grid=(grid,),
        in_specs=[
            pl.BlockSpec((blk, _D), lambda i: (i, 0)),
            pl.BlockSpec((blk, _D), lambda i: (i, 0)),
            pl.BlockSpec((blk, _D), lambda i: (i, 0)),
            pl.BlockSpec((blk, _D), lambda i: (i, 0)),
            pl.BlockSpec((blk, _D), lambda i: (i, 0)),
            pl.BlockSpec((_D, _D), lambda i: (0, 0)),
            pl.BlockSpec((_D, _D), lambda i: (0, 0)),
            pl.BlockSpec((1, _D), lambda i: (0, 0)),
        ],
        out_specs=pl.BlockSpec((blk, _D), lambda i: (i, 0)),
        out_shape=jax.ShapeDtypeStruct((_NP, _D), jnp.float32),
    )(h, a0, a1, c0, c1, Wl, Wr, b)


def _tc_pool_head_body(h_ref, brow_ref, bcol_ref, w1_ref, b1_ref, w2_ref,
                       b2_ref, o_ref, add_acc, max_acc, cnt_acc):
    i = pl.program_id(0)
    nsteps = pl.num_programs(0)

    @pl.when(i == 0)
    def _():
        add_acc[...] = jnp.zeros_like(add_acc)
        cnt_acc[...] = jnp.zeros_like(cnt_acc)
        max_acc[...] = jnp.full_like(max_acc, -3.0e38)

    h = h_ref[...]
    brow = brow_ref[0]
    onehot = jnp.where(
        brow == lax.broadcasted_iota(jnp.int32, (_G, brow.shape[1]), 0),
        1.0, 0.0)
    add_acc[...] += jnp.dot(onehot, h, preferred_element_type=jnp.float32)
    cnt_acc[...] += jnp.sum(onehot, axis=1, keepdims=True)

    bmin = brow[0, 0]
    bmax = brow[0, brow.shape[1] - 1]
    bcol = bcol_ref[...]
    for g in range(_G):
        @pl.when((jnp.int32(g) >= bmin) & (jnp.int32(g) <= bmax))
        def _(g=g):
            masked = jnp.where(bcol == g, h, -3.0e38)
            mx = jnp.max(masked, axis=0, keepdims=True)
            max_acc[g:g + 1, :] = jnp.maximum(max_acc[g:g + 1, :], mx)

    @pl.when(i == nsteps - 1)
    def _():
        cnt = cnt_acc[:, 0:1]
        meanp = add_acc[...] / jnp.maximum(cnt, 1.0)
        mx = jnp.where(cnt > 0, max_acc[...], 0.0)
        w1 = w1_ref[...]
        z = jnp.dot(add_acc[...], w1[0:_D, :],
                    preferred_element_type=jnp.float32)
        z += jnp.dot(meanp, w1[_D:2 * _D, :],
                     preferred_element_type=jnp.float32)
        z += jnp.dot(mx, w1[2 * _D:3 * _D, :],
                     preferred_element_type=jnp.float32)
        z = jnp.maximum(z + b1_ref[...], 0.0)
        o = jnp.dot(z, w2_ref[...], preferred_element_type=jnp.float32)
        o += b2_ref[...]
        m = jnp.max(o, axis=1, keepdims=True)
        sh = o - m
        lse = jnp.log(jnp.sum(jnp.exp(sh), axis=1, keepdims=True))
        o_ref[...] = sh - lse


def _tc_pool_head(h, brow, bcol, W1, b1, W2, b2):
    blk = 1024
    grid = _NP // blk
    return pl.pallas_call(
        _tc_pool_head_body,
        grid=(grid,),
        in_specs=[
            pl.BlockSpec((blk, _D), lambda i: (i, 0)),
            pl.BlockSpec((1, 1, blk), lambda i: (i, 0, 0)),
            pl.BlockSpec((blk, 1), lambda i: (i, 0)),
            pl.BlockSpec((3 * _D, _D), lambda i: (0, 0)),
            pl.BlockSpec((1, _D), lambda i: (0, 0)),
            pl.BlockSpec((_D, 2), lambda i: (0, 0)),
            pl.BlockSpec((1, 2), lambda i: (0, 0)),
        ],
        out_specs=pl.BlockSpec((_G, 2), lambda i: (0, 0)),
        out_shape=jax.ShapeDtypeStruct((_G, 2), jnp.float32),
        scratch_shapes=[
            pltpu.VMEM((_G, _D), jnp.float32),
            pltpu.VMEM((_G, _D), jnp.float32),
            pltpu.VMEM((_G, 1), jnp.float32),
        ],
    )(h, brow, bcol, W1, b1, W2, b2)


def kernel(x, edge_index, batch, Wl1, Wr1, b1, Wl2, Wr2, b2, Wl3, Wr3, b3,
           W_lin1, b_lin1, W_lin2, b_lin2):
    f32 = jnp.float32
    x = x.astype(f32)

    x_pad = jnp.zeros((_NP, _D), f32).at[:_N].set(x)
    src = edge_index[0].astype(jnp.int32)
    dst = edge_index[1].astype(jnp.int32)
    npad = _EP - _E
    src_p = jnp.concatenate([src, jnp.zeros((npad,), jnp.int32)])
    dst_p = jnp.concatenate([dst, jnp.full((npad,), _N, jnp.int32)])
    ei2d = jnp.stack([src_p.reshape(-1, _C), dst_p.reshape(-1, _C)],
                     axis=1).reshape(-1, _C)
    batch_p = jnp.concatenate(
        [batch.astype(jnp.int32), jnp.full((_NP - _N,), _G, jnp.int32)])
    brow = batch_p.reshape(_NP // 1024, 1, 1024)
    bcol = batch_p.reshape(_NP, 1)
    zerosZ = jnp.zeros((_ZR, _D), f32)
    zerosC = jnp.zeros((_C, _D), f32)
    onesC = jnp.ones((_C, _D), f32)
    b1r = b1.reshape(1, _D).astype(f32)
    b2r = b2.reshape(1, _D).astype(f32)
    b3r = b3.reshape(1, _D).astype(f32)
    bl1 = b_lin1.reshape(1, _D).astype(f32)
    bl2 = b_lin2.reshape(1, 2).astype(f32)

    cnt = _sc_degree(dst_p, zerosC, onesC)
    c0, c1 = cnt[0], cnt[1]

    agg = _sc_aggregate(x_pad, ei2d, zerosZ)
    h = _tc_layer(x_pad, agg[0], agg[1], c0, c1, Wl1, Wr1, b1r)

    agg = _sc_aggregate(h, ei2d, zerosZ)
    h = _tc_layer(h, agg[0], agg[1], c0, c1, Wl2, Wr2, b2r)

    agg = _sc_aggregate(h, ei2d, zerosZ)
    h = _tc_layer(h, agg[0], agg[1], c0, c1, Wl3, Wr3, b3r)

    return _tc_pool_head(h, brow, bcol, W_lin1, bl1, W_lin2, bl2)
